# baseline (speedup 1.0000x reference)
import functools

import jax
import jax.numpy as jnp
from jax import lax
from jax.experimental import pallas as pl
from jax.experimental.pallas import tpu as pltpu
from jax.experimental.pallas import tpu_sc as plsc

B, N, D, K = 4, 8192, 2048, 256
BN = 256
NSTEPS = N // BN


def _score_topk_body(x_ref, w_ref, b_ref, probs_ref, idx_ref, flat_ref,
                     scores_scr):
    j = pl.program_id(0)
    xb = x_ref[...].astype(jnp.bfloat16).astype(jnp.float32)
    wv = w_ref[...].astype(jnp.bfloat16).astype(jnp.float32)
    s = jnp.sum(xb * wv, axis=-1) + b_ref[0, 0]
    scores_scr[:, pl.ds(j * BN, BN)] = s

    @pl.when(j == NSTEPS - 1)
    def _finalize():
        scores = scores_scr[...]
        m = jnp.max(scores, axis=1, keepdims=True)
        u = jnp.exp(scores - m)
        ssum = jnp.sum(u, axis=1, keepdims=True)
        p = u / ssum
        probs_ref[...] = p

        iota = lax.broadcasted_iota(jnp.int32, (B, N), 1)
        iota_k = lax.broadcasted_iota(jnp.int32, (B, K), 1)
        boff = lax.broadcasted_iota(jnp.int32, (B, K), 0) * N

        def body(t, carry):
            vals, idxs = carry
            mx = jnp.max(vals, axis=1, keepdims=True)
            tie = vals == mx
            idx = jnp.min(jnp.where(tie, iota, N), axis=1, keepdims=True)
            idxs = jnp.where(iota_k == t, idx, idxs)
            vals = jnp.where(iota == idx, -jnp.inf, vals)
            return (vals, idxs)

        idxs0 = jnp.zeros((B, K), jnp.int32)
        _, idxs = lax.fori_loop(0, K, body, (p, idxs0))
        idx_ref[...] = idxs
        flat_ref[...] = idxs + boff


_score_topk = pl.pallas_call(
    _score_topk_body,
    grid=(NSTEPS,),
    in_specs=[
        pl.BlockSpec((B, BN, D), lambda j: (0, j, 0)),
        pl.BlockSpec((D,), lambda j: (0,)),
        pl.BlockSpec(memory_space=pltpu.SMEM),
    ],
    out_specs=[
        pl.BlockSpec((B, N), lambda j: (0, 0)),
        pl.BlockSpec((B, K), lambda j: (0, 0)),
        pl.BlockSpec((B, K), lambda j: (0, 0)),
    ],
    out_shape=[
        jax.ShapeDtypeStruct((B, N), jnp.float32),
        jax.ShapeDtypeStruct((B, K), jnp.int32),
        jax.ShapeDtypeStruct((B, K), jnp.int32),
    ],
    scratch_shapes=[pltpu.VMEM((B, N), jnp.float32)],
)


_NC = 2
_NS = 16
_NW = _NC * _NS
_ROWS = B * K
_RPW = _ROWS // _NW


def _gather_body(x_hbm, idx_hbm, out_hbm, idx_v, rows_v, sem):
    wid = lax.axis_index("s") * _NC + lax.axis_index("c")
    base = wid * _RPW
    pltpu.sync_copy(idx_hbm.at[pl.ds(base, _RPW)], idx_v)
    pltpu.async_copy(x_hbm.at[idx_v], rows_v, sem).wait()
    pltpu.sync_copy(rows_v, out_hbm.at[pl.ds(base, _RPW)])


@functools.lru_cache(maxsize=None)
def _make_gather():
    return pl.kernel(
        _gather_body,
        out_type=jax.ShapeDtypeStruct((_ROWS, D), jnp.float32),
        mesh=plsc.VectorSubcoreMesh(core_axis_name="c", subcore_axis_name="s",
                                    num_cores=_NC, num_subcores=_NS),
        scratch_types=[
            pltpu.VMEM((_RPW,), jnp.int32),
            pltpu.VMEM((_RPW, D), jnp.float32),
            pltpu.SemaphoreType.DMA,
        ],
    )


def kernel(x, W, b):
    probs, idx, flat = _score_topk(x, W, jnp.asarray(b).reshape(1, 1))
    rows = _make_gather()(x.reshape(B * N, D), flat.reshape(_ROWS))
    selected = rows.reshape(B, K, D)
    return (selected, probs, idx)
